# baseline (speedup 1.0000x reference)
import functools

import jax
import jax.numpy as jnp
from jax import lax
from jax.experimental import pallas as pl
from jax.experimental.pallas import tpu as pltpu
from jax.experimental.pallas import tpu_sc as plsc

N = 425984
D = 32
NC = 2
NS = 16
NW = NC * NS
B_PER_W = N // NW
CHUNK = 1664
NCHUNK = B_PER_W // CHUNK


@functools.partial(
    pl.kernel,
    out_type=jax.ShapeDtypeStruct((N, D), jnp.float32),
    mesh=plsc.VectorSubcoreMesh(core_axis_name="c", subcore_axis_name="s"),
    scratch_types=[
        pltpu.VMEM((CHUNK,), jnp.int32),
        pltpu.VMEM((CHUNK,), jnp.int32),
        pltpu.VMEM((CHUNK, D), jnp.float32),
        pltpu.SemaphoreType.DMA,
    ],
    compiler_params=pltpu.CompilerParams(use_tc_tiling_on_sc=False),
)
def _bi_embed(kidx_hbm, vidx_hbm, keys_hbm, values_hbm, out_hbm,
              kidx_v, vidx_v, rows_v, sem):
    wid = lax.axis_index("s") * NC + lax.axis_index("c")
    base = pl.multiple_of(wid * B_PER_W, CHUNK)

    def body(ci, _):
        off = pl.multiple_of(base + ci * CHUNK, CHUNK)
        pltpu.sync_copy(kidx_hbm.at[pl.ds(off, CHUNK)], kidx_v)
        pltpu.sync_copy(vidx_hbm.at[pl.ds(off, CHUNK)], vidx_v)
        pltpu.async_copy(keys_hbm.at[kidx_v], rows_v, sem).wait()
        pltpu.async_copy(values_hbm.at[vidx_v], rows_v, sem, add=True).wait()
        pltpu.sync_copy(rows_v, out_hbm.at[pl.ds(off, CHUNK)])
        return 0

    lax.fori_loop(0, NCHUNK, body, 0)


def kernel(data, keys_table, values_table):
    kidx = data[:, 0]
    vidx = data[:, 1]
    return _bi_embed(kidx, vidx, keys_table, values_table)


# speedup vs baseline: 1.0139x; 1.0139x over previous
import functools

import jax
import jax.numpy as jnp
from jax import lax
from jax.experimental import pallas as pl
from jax.experimental.pallas import tpu as pltpu
from jax.experimental.pallas import tpu_sc as plsc

N = 425984
D = 32
NC = 2
NS = 16
NW = NC * NS
B_PER_W = N // NW
CHUNK = 832
NCHUNK = B_PER_W // CHUNK
NBUF = 3


@functools.partial(
    pl.kernel,
    out_type=jax.ShapeDtypeStruct((N, D), jnp.float32),
    mesh=plsc.VectorSubcoreMesh(core_axis_name="c", subcore_axis_name="s"),
    scratch_types=[
        pltpu.VMEM((B_PER_W,), jnp.int32),
        pltpu.VMEM((B_PER_W,), jnp.int32),
        pltpu.VMEM((NBUF, CHUNK, D), jnp.float32),
        pltpu.SemaphoreType.DMA((NBUF,)),
        pltpu.SemaphoreType.DMA((NBUF,)),
        pltpu.SemaphoreType.DMA((NBUF,)),
    ],
    compiler_params=pltpu.CompilerParams(use_tc_tiling_on_sc=False),
)
def _bi_embed(kidx_hbm, vidx_hbm, keys_hbm, values_hbm, out_hbm,
              kidx_v, vidx_v, rows_v, semk, semv, semo):
    wid = lax.axis_index("s") * NC + lax.axis_index("c")
    base = pl.multiple_of(wid * B_PER_W, CHUNK)

    pltpu.sync_copy(kidx_hbm.at[pl.ds(base, B_PER_W)], kidx_v)
    pltpu.sync_copy(vidx_hbm.at[pl.ds(base, B_PER_W)], vidx_v)

    def gather_k(ci, b):
        src = keys_hbm.at[kidx_v.at[pl.ds(ci * CHUNK, CHUNK)]]
        return pltpu.make_async_copy(src, rows_v.at[b], semk.at[b])

    def gather_v_start(ci, b):
        src = values_hbm.at[vidx_v.at[pl.ds(ci * CHUNK, CHUNK)]]
        pltpu.async_copy(src, rows_v.at[b], semv.at[b], add=True)

    def gather_v_wait(ci, b):
        src = values_hbm.at[vidx_v.at[pl.ds(ci * CHUNK, CHUNK)]]
        pltpu.make_async_copy(src, rows_v.at[b], semv.at[b]).wait()

    def writeback(ci, b):
        off = pl.multiple_of(base + ci * CHUNK, CHUNK)
        return pltpu.make_async_copy(
            rows_v.at[b], out_hbm.at[pl.ds(off, CHUNK)], semo.at[b])

    def body(i, _):
        b0 = lax.rem(i, NBUF)
        j = i - 1
        k = i - 2
        b1 = lax.rem(j + NBUF, NBUF)
        b2 = lax.rem(k + NBUF, NBUF)

        @pl.when(i >= NBUF)
        def _():
            writeback(i - NBUF, b0).wait()

        @pl.when(i < NCHUNK)
        def _():
            gather_k(i, b0).start()

        @pl.when(jnp.logical_and(j >= 0, j < NCHUNK))
        def _():
            gather_k(j, b1).wait()
            gather_v_start(j, b1)

        @pl.when(jnp.logical_and(k >= 0, k < NCHUNK))
        def _():
            gather_v_wait(k, b2)
            writeback(k, b2).start()

        return 0

    lax.fori_loop(0, NCHUNK + 2, body, 0)

    writeback(NCHUNK - 1, (NCHUNK - 1) % NBUF).wait()


def kernel(data, keys_table, values_table):
    kidx = data[:, 0]
    vidx = data[:, 1]
    return _bi_embed(kidx, vidx, keys_table, values_table)
